# baseline (device time: 866633 ns/iter reference)
import jax
import jax.numpy as jnp
from jax import lax
from jax.experimental import pallas as pl
from jax.experimental.pallas import tpu as pltpu

T_CHUNK = 128
PAIR = 2 * T_CHUNK
RECV_SLOTS = 2
W_TILE = 256
PAD = 128


def kernel(x, W):
    t, d = x.shape
    _, v = W.shape
    n = t // T_CHUNK
    npairs = n // 2
    C = v // W_TILE
    G = npairs * C
    vp = v + PAD

    def body(
        x_ref, w_ref, out_ref,
        wc, xp, loc, recv, osl,
        wsems, xsems, send_sems, recv_sems, out_sem, credit_sem,
    ):
        my_x = lax.axis_index("x")
        my_y = lax.axis_index("y")
        my_z = lax.axis_index("z")
        partner = (1 - my_x, my_y, my_z)
        T = T_CHUNK

        def rem(a, k):
            return a % k if isinstance(a, int) else lax.rem(a, k)

        def div(a, k):
            return a // k if isinstance(a, int) else lax.div(a, k)

        def wdma(g):
            return pltpu.make_async_copy(
                w_ref.at[:, pl.ds(rem(g, C) * W_TILE, W_TILE)],
                wc.at[rem(g, 2)],
                wsems.at[rem(g, 2)],
            )

        def xpl(q):
            return pltpu.make_async_copy(
                x_ref.at[pl.ds(q * PAIR, PAIR), :],
                xp.at[rem(q, 2)],
                xsems.at[rem(q, 2)],
            )

        def rdma(j):
            return pltpu.make_async_remote_copy(
                src_ref=loc.at[rem(div(j, 2), 2), pl.ds(rem(j, 2) * T, T)],
                dst_ref=recv.at[rem(j, RECV_SLOTS)],
                send_sem=send_sems.at[rem(j, 2)],
                recv_sem=recv_sems.at[rem(j, RECV_SLOTS)],
                device_id=partner,
                device_id_type=pl.DeviceIdType.MESH,
            )

        def out_dma(j):
            return pltpu.make_async_copy(
                osl, out_ref.at[pl.ds(j * T, T), :], out_sem
            )

        def gemm_pair(q):
            ps = rem(q, 2)
            base = q * C

            def tile(c, s_acc):
                g = base + c
                wdma(g).wait()

                @pl.when(g + 2 < G)
                def _():
                    wdma(g + 2).start()

                el = jnp.exp(
                    jnp.dot(
                        xp[rem(q, 2)], wc[rem(g, 2)],
                        preferred_element_type=jnp.float32,
                    )
                )
                loc[ps, :, pl.ds(c * W_TILE, W_TILE)] = el
                return s_acc + jnp.sum(el, axis=-1, keepdims=True)

            s = lax.fori_loop(
                0, C, tile, jnp.zeros((PAIR, 1), jnp.float32)
            )
            loc[ps, :, pl.ds(v, PAD)] = jnp.broadcast_to(s, (PAIR, PAD))

        xpl(0).start()
        barrier = pltpu.get_barrier_semaphore()
        pl.semaphore_signal(
            barrier, inc=1, device_id=partner,
            device_id_type=pl.DeviceIdType.MESH,
        )
        pl.semaphore_wait(barrier, 1)
        wdma(0).start()
        wdma(1).start()
        xpl(0).wait()
        gemm_pair(0)
        xpl(1).start()
        rdma(0).start()

        def step(i, _):
            @pl.when(i + 1 < n)
            def _():
                @pl.when(i + 1 >= 2)
                def _():
                    rdma(i - 1).wait_send()

                @pl.when(i + 1 >= RECV_SLOTS)
                def _():
                    pl.semaphore_wait(credit_sem, 1)

                rdma(i + 1).start()

            @pl.when((rem(i, 2) == 0) & (i + 2 < n))
            def _():
                q = div(i, 2) + 1
                xpl(q).wait()
                gemm_pair(q)

                @pl.when(q + 1 < npairs)
                def _():
                    xpl(q + 1).start()

            rdma(i).wait_recv()

            @pl.when(i >= 1)
            def _():
                out_dma(i - 1).wait()

            lo = loc[rem(div(i, 2), 2), pl.ds(rem(i, 2) * T, T)]
            rm = recv[rem(i, RECV_SLOTS)]
            inv = 1.0 / (lo[:, v:v + 1] + rm[:, v:v + 1])
            osl[:, pl.ds(my_x * v, v)] = lo[:, :v] * inv
            osl[:, pl.ds((1 - my_x) * v, v)] = rm[:, :v] * inv
            pl.semaphore_signal(
                credit_sem, inc=1, device_id=partner,
                device_id_type=pl.DeviceIdType.MESH,
            )
            out_dma(i).start()
            return 0

        lax.fori_loop(0, n, step, 0)

        rdma(n - 2).wait_send()
        rdma(n - 1).wait_send()
        out_dma(n - 1).wait()
        pl.semaphore_wait(credit_sem, RECV_SLOTS)

    return pl.pallas_call(
        body,
        in_specs=[
            pl.BlockSpec(memory_space=pl.ANY),
            pl.BlockSpec(memory_space=pl.ANY),
        ],
        out_specs=pl.BlockSpec(memory_space=pl.ANY),
        out_shape=jax.ShapeDtypeStruct((t, 2 * v), jnp.float32),
        scratch_shapes=[
            pltpu.VMEM((2, d, W_TILE), jnp.float32),
            pltpu.VMEM((2, PAIR, d), jnp.float32),
            pltpu.VMEM((2, PAIR, vp), jnp.float32),
            pltpu.VMEM((RECV_SLOTS, T_CHUNK, vp), jnp.float32),
            pltpu.VMEM((T_CHUNK, 2 * v), jnp.float32),
            pltpu.SemaphoreType.DMA((2,)),
            pltpu.SemaphoreType.DMA((2,)),
            pltpu.SemaphoreType.DMA((2,)),
            pltpu.SemaphoreType.DMA((RECV_SLOTS,)),
            pltpu.SemaphoreType.DMA,
            pltpu.SemaphoreType.REGULAR,
        ],
        compiler_params=pltpu.CompilerParams(
            collective_id=0, vmem_limit_bytes=63 * 1024 * 1024
        ),
    )(x, W)


# device time: 838358 ns/iter; 1.0337x vs baseline; 1.0337x over previous
import jax
import jax.numpy as jnp
from jax import lax
from jax.experimental import pallas as pl
from jax.experimental.pallas import tpu as pltpu

T_CHUNK = 128
PAIR = 2 * T_CHUNK
RECV_SLOTS = 2
W_TILE = 256
PAD = 128
NSUB = 4


def kernel(x, W):
    t, d = x.shape
    _, v = W.shape
    n = t // T_CHUNK
    npairs = n // 2
    C = v // W_TILE
    G = npairs * C
    vp = v + PAD
    QW = C // NSUB
    SUBV = v // NSUB

    def body(
        x_ref, w_ref, out_ref,
        wc, xp, loc, recv, osl,
        wsems, xsems, send_sems, recv_sems, out_sem, credit_sem,
        fsends, frecvs,
    ):
        my_x = lax.axis_index("x")
        my_y = lax.axis_index("y")
        my_z = lax.axis_index("z")
        partner = (1 - my_x, my_y, my_z)
        T = T_CHUNK

        def rem(a, k):
            return a % k if isinstance(a, int) else lax.rem(a, k)

        def div(a, k):
            return a // k if isinstance(a, int) else lax.div(a, k)

        def wdma(g):
            return pltpu.make_async_copy(
                w_ref.at[:, pl.ds(rem(g, C) * W_TILE, W_TILE)],
                wc.at[rem(g, 2)],
                wsems.at[rem(g, 2)],
            )

        def xpl(q):
            return pltpu.make_async_copy(
                x_ref.at[pl.ds(q * PAIR, PAIR), :],
                xp.at[rem(q, 2)],
                xsems.at[rem(q, 2)],
            )

        def rdma(j):
            return pltpu.make_async_remote_copy(
                src_ref=loc.at[rem(div(j, 2), 2), pl.ds(rem(j, 2) * T, T)],
                dst_ref=recv.at[rem(j, RECV_SLOTS)],
                send_sem=send_sems.at[rem(j, 2)],
                recv_sem=recv_sems.at[rem(j, RECV_SLOTS)],
                device_id=partner,
                device_id_type=pl.DeviceIdType.MESH,
            )

        def sub_rdma(k):
            w = SUBV if k < NSUB - 1 else SUBV + PAD
            return pltpu.make_async_remote_copy(
                src_ref=loc.at[0, pl.ds(0, T), pl.ds(k * SUBV, w)],
                dst_ref=recv.at[0, pl.ds(0, T), pl.ds(k * SUBV, w)],
                send_sem=fsends.at[k],
                recv_sem=frecvs.at[k],
                device_id=partner,
                device_id_type=pl.DeviceIdType.MESH,
            )

        def out_dma(j):
            return pltpu.make_async_copy(
                osl, out_ref.at[pl.ds(j * T, T), :], out_sem
            )

        def make_tile(q, ps, first):
            def tile(c, s_acc):
                g = q * C + c
                wdma(g).wait()

                @pl.when(g + 2 < G)
                def _():
                    wdma(g + 2).start()

                el = jnp.exp(
                    jnp.dot(
                        xp[rem(q, 2)], wc[rem(g, 2)],
                        preferred_element_type=jnp.float32,
                    )
                )
                loc[ps, :, pl.ds(c * W_TILE, W_TILE)] = el
                if first:
                    for k in range(NSUB - 1):
                        @pl.when(c == (k + 1) * QW - 1)
                        def _(k=k):
                            sub_rdma(k).start()
                return s_acc + jnp.sum(el, axis=-1, keepdims=True)

            return tile

        def gemm_pair(q, first=False):
            ps = rem(q, 2)
            s = lax.fori_loop(
                0, C, make_tile(q, ps, first),
                jnp.zeros((PAIR, 1), jnp.float32),
            )
            loc[ps, :, pl.ds(v, PAD)] = jnp.broadcast_to(s, (PAIR, PAD))
            if first:
                sub_rdma(NSUB - 1).start()

        xpl(0).start()
        barrier = pltpu.get_barrier_semaphore()
        pl.semaphore_signal(
            barrier, inc=1, device_id=partner,
            device_id_type=pl.DeviceIdType.MESH,
        )
        pl.semaphore_wait(barrier, 1)
        wdma(0).start()
        wdma(1).start()
        xpl(0).wait()
        gemm_pair(0, first=True)
        xpl(1).start()

        def step(i, _):
            @pl.when(i + 1 < n)
            def _():
                @pl.when(i + 1 >= 3)
                def _():
                    rdma(i - 1).wait_send()

                @pl.when(i + 1 >= RECV_SLOTS)
                def _():
                    pl.semaphore_wait(credit_sem, 1)

                rdma(i + 1).start()

            @pl.when((rem(i, 2) == 0) & (i + 2 < n))
            def _():
                q = div(i, 2) + 1
                xpl(q).wait()
                gemm_pair(q)

                @pl.when(q + 1 < npairs)
                def _():
                    xpl(q + 1).start()

            @pl.when(i == 0)
            def _():
                for k in range(NSUB):
                    sub_rdma(k).wait_recv()

            @pl.when(i > 0)
            def _():
                rdma(i).wait_recv()

            @pl.when(i >= 1)
            def _():
                out_dma(i - 1).wait()

            lo = loc[rem(div(i, 2), 2), pl.ds(rem(i, 2) * T, T)]
            rm = recv[rem(i, RECV_SLOTS)]
            inv = 1.0 / (lo[:, v:v + 1] + rm[:, v:v + 1])
            osl[:, pl.ds(my_x * v, v)] = lo[:, :v] * inv
            osl[:, pl.ds((1 - my_x) * v, v)] = rm[:, :v] * inv
            pl.semaphore_signal(
                credit_sem, inc=1, device_id=partner,
                device_id_type=pl.DeviceIdType.MESH,
            )
            out_dma(i).start()
            return 0

        lax.fori_loop(0, n, step, 0)

        for k in range(NSUB):
            sub_rdma(k).wait_send()
        rdma(n - 2).wait_send()
        rdma(n - 1).wait_send()
        out_dma(n - 1).wait()
        pl.semaphore_wait(credit_sem, RECV_SLOTS)

    return pl.pallas_call(
        body,
        in_specs=[
            pl.BlockSpec(memory_space=pl.ANY),
            pl.BlockSpec(memory_space=pl.ANY),
        ],
        out_specs=pl.BlockSpec(memory_space=pl.ANY),
        out_shape=jax.ShapeDtypeStruct((t, 2 * v), jnp.float32),
        scratch_shapes=[
            pltpu.VMEM((2, d, W_TILE), jnp.float32),
            pltpu.VMEM((2, PAIR, d), jnp.float32),
            pltpu.VMEM((2, PAIR, vp), jnp.float32),
            pltpu.VMEM((RECV_SLOTS, T_CHUNK, vp), jnp.float32),
            pltpu.VMEM((T_CHUNK, 2 * v), jnp.float32),
            pltpu.SemaphoreType.DMA((2,)),
            pltpu.SemaphoreType.DMA((2,)),
            pltpu.SemaphoreType.DMA((2,)),
            pltpu.SemaphoreType.DMA((RECV_SLOTS,)),
            pltpu.SemaphoreType.DMA,
            pltpu.SemaphoreType.REGULAR,
            pltpu.SemaphoreType.DMA((NSUB,)),
            pltpu.SemaphoreType.DMA((NSUB,)),
        ],
        compiler_params=pltpu.CompilerParams(
            collective_id=0, vmem_limit_bytes=63 * 1024 * 1024
        ),
    )(x, W)


# device time: 831491 ns/iter; 1.0423x vs baseline; 1.0083x over previous
import jax
import jax.numpy as jnp
from jax import lax
from jax.experimental import pallas as pl
from jax.experimental.pallas import tpu as pltpu

T_CHUNK = 128
PAIR = 2 * T_CHUNK
RECV_SLOTS = 2
W_TILE = 256
PAD = 128
NSUB = 8


def kernel(x, W):
    t, d = x.shape
    _, v = W.shape
    n = t // T_CHUNK
    npairs = n // 2
    C = v // W_TILE
    G = npairs * C
    vp = v + PAD
    QW = C // NSUB
    SUBV = v // NSUB

    def body(
        x_ref, w_ref, out_ref,
        wc, xp, loc, recv, osl,
        wsems, xsems, send_sems, recv_sems, out_sem, credit_sem,
        fsends, frecvs,
    ):
        my_x = lax.axis_index("x")
        my_y = lax.axis_index("y")
        my_z = lax.axis_index("z")
        partner = (1 - my_x, my_y, my_z)
        T = T_CHUNK

        def rem(a, k):
            return a % k if isinstance(a, int) else lax.rem(a, k)

        def div(a, k):
            return a // k if isinstance(a, int) else lax.div(a, k)

        def wdma(g):
            return pltpu.make_async_copy(
                w_ref.at[:, pl.ds(rem(g, C) * W_TILE, W_TILE)],
                wc.at[rem(g, 2)],
                wsems.at[rem(g, 2)],
            )

        def xpl(q):
            return pltpu.make_async_copy(
                x_ref.at[pl.ds(q * PAIR, PAIR), :],
                xp.at[rem(q, 2)],
                xsems.at[rem(q, 2)],
            )

        def rdma(j):
            return pltpu.make_async_remote_copy(
                src_ref=loc.at[rem(div(j, 2), 2), pl.ds(rem(j, 2) * T, T)],
                dst_ref=recv.at[rem(j, RECV_SLOTS)],
                send_sem=send_sems.at[rem(j, 2)],
                recv_sem=recv_sems.at[rem(j, RECV_SLOTS)],
                device_id=partner,
                device_id_type=pl.DeviceIdType.MESH,
            )

        def sub_rdma(k):
            w = SUBV if k < NSUB - 1 else SUBV + PAD
            return pltpu.make_async_remote_copy(
                src_ref=loc.at[0, pl.ds(0, T), pl.ds(k * SUBV, w)],
                dst_ref=recv.at[0, pl.ds(0, T), pl.ds(k * SUBV, w)],
                send_sem=fsends.at[k],
                recv_sem=frecvs.at[k],
                device_id=partner,
                device_id_type=pl.DeviceIdType.MESH,
            )

        def out_dma(j):
            return pltpu.make_async_copy(
                osl, out_ref.at[pl.ds(j * T, T), :], out_sem
            )

        def make_tile(q, ps, first):
            def tile(c, s_acc):
                g = q * C + c
                wdma(g).wait()

                @pl.when(g + 2 < G)
                def _():
                    wdma(g + 2).start()

                el = jnp.exp(
                    jnp.dot(
                        xp[rem(q, 2)], wc[rem(g, 2)],
                        preferred_element_type=jnp.float32,
                    )
                )
                loc[ps, :, pl.ds(c * W_TILE, W_TILE)] = el
                if first:
                    for k in range(NSUB - 1):
                        @pl.when(c == (k + 1) * QW - 1)
                        def _(k=k):
                            sub_rdma(k).start()
                return s_acc + jnp.sum(el, axis=-1, keepdims=True)

            return tile

        def gemm_pair(q, first=False):
            ps = rem(q, 2)
            s = lax.fori_loop(
                0, C, make_tile(q, ps, first),
                jnp.zeros((PAIR, 1), jnp.float32),
            )
            loc[ps, :, pl.ds(v, PAD)] = jnp.broadcast_to(s, (PAIR, PAD))
            if first:
                sub_rdma(NSUB - 1).start()

        xpl(0).start()
        barrier = pltpu.get_barrier_semaphore()
        pl.semaphore_signal(
            barrier, inc=1, device_id=partner,
            device_id_type=pl.DeviceIdType.MESH,
        )
        pl.semaphore_wait(barrier, 1)
        wdma(0).start()
        wdma(1).start()
        xpl(0).wait()
        gemm_pair(0, first=True)
        xpl(1).start()

        def step(i, _):
            @pl.when(i + 1 < n)
            def _():
                @pl.when(i + 1 >= 3)
                def _():
                    rdma(i - 1).wait_send()

                @pl.when(i + 1 >= RECV_SLOTS)
                def _():
                    pl.semaphore_wait(credit_sem, 1)

                rdma(i + 1).start()

            @pl.when((rem(i, 2) == 0) & (i + 2 < n))
            def _():
                q = div(i, 2) + 1
                xpl(q).wait()
                gemm_pair(q)

                @pl.when(q + 1 < npairs)
                def _():
                    xpl(q + 1).start()

            @pl.when(i == 0)
            def _():
                for k in range(NSUB):
                    sub_rdma(k).wait_recv()

            @pl.when(i > 0)
            def _():
                rdma(i).wait_recv()

            @pl.when(i >= 1)
            def _():
                out_dma(i - 1).wait()

            lo = loc[rem(div(i, 2), 2), pl.ds(rem(i, 2) * T, T)]
            rm = recv[rem(i, RECV_SLOTS)]
            inv = 1.0 / (lo[:, v:v + 1] + rm[:, v:v + 1])
            osl[:, pl.ds(my_x * v, v)] = lo[:, :v] * inv
            osl[:, pl.ds((1 - my_x) * v, v)] = rm[:, :v] * inv
            pl.semaphore_signal(
                credit_sem, inc=1, device_id=partner,
                device_id_type=pl.DeviceIdType.MESH,
            )
            out_dma(i).start()
            return 0

        lax.fori_loop(0, n, step, 0)

        for k in range(NSUB):
            sub_rdma(k).wait_send()
        rdma(n - 2).wait_send()
        rdma(n - 1).wait_send()
        out_dma(n - 1).wait()
        pl.semaphore_wait(credit_sem, RECV_SLOTS)

    return pl.pallas_call(
        body,
        in_specs=[
            pl.BlockSpec(memory_space=pl.ANY),
            pl.BlockSpec(memory_space=pl.ANY),
        ],
        out_specs=pl.BlockSpec(memory_space=pl.ANY),
        out_shape=jax.ShapeDtypeStruct((t, 2 * v), jnp.float32),
        scratch_shapes=[
            pltpu.VMEM((2, d, W_TILE), jnp.float32),
            pltpu.VMEM((2, PAIR, d), jnp.float32),
            pltpu.VMEM((2, PAIR, vp), jnp.float32),
            pltpu.VMEM((RECV_SLOTS, T_CHUNK, vp), jnp.float32),
            pltpu.VMEM((T_CHUNK, 2 * v), jnp.float32),
            pltpu.SemaphoreType.DMA((2,)),
            pltpu.SemaphoreType.DMA((2,)),
            pltpu.SemaphoreType.DMA((2,)),
            pltpu.SemaphoreType.DMA((RECV_SLOTS,)),
            pltpu.SemaphoreType.DMA,
            pltpu.SemaphoreType.REGULAR,
            pltpu.SemaphoreType.DMA((NSUB,)),
            pltpu.SemaphoreType.DMA((NSUB,)),
        ],
        compiler_params=pltpu.CompilerParams(
            collective_id=0, vmem_limit_bytes=63 * 1024 * 1024
        ),
    )(x, W)
